# baseline (speedup 1.0000x reference)
import functools

import jax
import jax.numpy as jnp
from jax import lax
from jax.experimental import pallas as pl
from jax.experimental.pallas import tpu as pltpu
from jax.experimental.pallas import tpu_sc as plsc

B = 4096
L = 50
DIM = 64
N = B * L
V = 1_000_000


CB = 4096


NBLK = pl.cdiv(V, CB)
VPAIR = NBLK * (CB // 2)


def _tc_repack(tt_ref, out_ref):
  x = tt_ref[...]
  y = jnp.swapaxes(x, 0, 1)
  out_ref[...] = jnp.concatenate([y[: CB // 2], y[CB // 2 :]], axis=1)


def _repack(table_t):
  return pl.pallas_call(
      _tc_repack,
      grid=(NBLK,),
      in_specs=[pl.BlockSpec((DIM, CB), lambda i: (0, i))],
      out_specs=pl.BlockSpec((CB // 2, 128), lambda i: (i, 0)),
      out_shape=jax.ShapeDtypeStruct((VPAIR, 128), jnp.float32),
  )(table_t)


NC = 2
NS = 16
NW = NC * NS
B_PER_W = B // NW
RPW = B_PER_W * L
CHUNK = 160
NCHUNK = RPW // CHUNK
NSUPER = NCHUNK // 2
NBUF = 2
KG = CHUNK // 2 // 16


def _sc_gather(
    idx_hbm, pairs_hbm, out_hbm, idx_v, glist0_v, glist1_v, sel_v, raw_v,
    stage_v, gsem, osem
):
  glists = (glist0_v, glist1_v)
  wid = lax.axis_index("s") * NC + lax.axis_index("c")
  b0 = wid * B_PER_W
  pltpu.sync_copy(idx_hbm.at[pl.ds(b0, B_PER_W)], idx_v)

  iota = lax.iota(jnp.int32, 16)

  def build_lists(g, buf):
    def step(t, _):
      p = g * CHUNK + t * 16 + iota
      r = lax.div(p, jnp.int32(L))
      c = p - r * L
      v = plsc.load_gather(idx_v, [r, c])
      bufv = jnp.int32(buf) + 0 * iota
      pr = lax.shift_right_logical(v, 12) * (CB // 2) + lax.bitwise_and(
          v, CB // 2 - 1
      )
      plsc.store_scatter(glists[buf], [t * 16 + iota], pr)
      plsc.store_scatter(
          sel_v,
          [bufv, t * 16 + iota],
          lax.shift_left(
              lax.bitwise_and(lax.shift_right_logical(v, 11), 1), 6
          ),
      )
      return 0

    lax.fori_loop(0, CHUNK // 16, step, 0)

  def gather_start(g, buf):
    build_lists(g, buf)
    pltpu.async_copy(pairs_hbm.at[glists[buf]], raw_v.at[buf], gsem)

  def gather_wait(buf):
    pltpu.make_async_copy(
        pairs_hbm.at[glists[buf]], raw_v.at[buf], gsem
    ).wait()

  def select(buf):
    def grp_step(t, _):
      offv = sel_v[buf, pl.ds(16 * t, 16)]
      for i in range(16):
        r = 16 * t + i
        off = offv[i]
        for q in range(DIM // 16):
          stage_v[buf, 8 * t + i // 2, pl.ds((i % 2) * DIM + 16 * q, 16)] = (
              raw_v[buf, r, pl.ds(off + 16 * q, 16)]
          )
      return 0

    lax.fori_loop(0, CHUNK // 16, grp_step, 0)

  def out_start(g, buf):
    pltpu.async_copy(
        stage_v.at[buf],
        out_hbm.at[pl.ds(wid * (RPW // 2) + g * (CHUNK // 2), CHUNK // 2)],
        osem,
    )

  def out_wait(buf):
    pltpu.make_async_copy(
        stage_v.at[buf],
        out_hbm.at[pl.ds(wid * (RPW // 2), CHUNK // 2)],
        osem,
    ).wait()

  gather_start(0, 0)

  def super_step(s, _):
    g0 = 2 * s
    gather_wait(0)
    gather_start(g0 + 1, 1)

    @pl.when(s >= 1)
    def _():
      out_wait(0)

    select(0)
    out_start(g0, 0)

    gather_wait(1)

    @pl.when(s + 1 < NSUPER)
    def _():
      gather_start(g0 + 2, 0)

    @pl.when(s >= 1)
    def _():
      out_wait(1)

    select(1)
    out_start(g0 + 1, 1)
    return 0

  lax.fori_loop(0, NSUPER, super_step, 0)
  out_wait(0)
  out_wait(1)


@jax.jit
def _embedding(idx2d, table):
  pairs = _repack(table.T)
  mesh = plsc.VectorSubcoreMesh(core_axis_name="c", subcore_axis_name="s")
  f = pl.kernel(
      _sc_gather,
      out_type=jax.ShapeDtypeStruct((N // 2, 128), jnp.float32),
      mesh=mesh,
      scratch_types=[
          pltpu.VMEM((B_PER_W, L), jnp.int32),
          pltpu.VMEM((CHUNK,), jnp.int32),
          pltpu.VMEM((CHUNK,), jnp.int32),
          pltpu.VMEM((NBUF, CHUNK), jnp.int32),
          pltpu.VMEM((NBUF, CHUNK, 128), jnp.float32),
          pltpu.VMEM((NBUF, CHUNK // 2, 128), jnp.float32),
          pltpu.SemaphoreType.DMA,
          pltpu.SemaphoreType.DMA,
      ],
      compiler_params=pltpu.CompilerParams(
          use_tc_tiling_on_sc=True, needs_layout_passes=False
      ),
  )
  out_pairs = f(idx2d, pairs)
  return out_pairs.reshape(B, L, DIM)


def kernel(input, table):
  return _embedding(input.astype(jnp.int32), table)


# speedup vs baseline: 1.1261x; 1.1261x over previous
import functools

import jax
import jax.numpy as jnp
from jax import lax
from jax.experimental import pallas as pl
from jax.experimental.pallas import tpu as pltpu
from jax.experimental.pallas import tpu_sc as plsc

B = 4096
L = 50
DIM = 64
N = B * L
V = 1_000_000


CB = 8192


NBLK = pl.cdiv(V, CB)
VPAIR = NBLK * (CB // 2)


def _tc_repack(tt_ref, out_ref):
  x = tt_ref[...]
  y = jnp.swapaxes(x, 0, 1)
  out_ref[...] = jnp.concatenate([y[: CB // 2], y[CB // 2 :]], axis=1)


def _repack(table_t):
  return pl.pallas_call(
      _tc_repack,
      grid=(NBLK,),
      in_specs=[pl.BlockSpec((DIM, CB), lambda i: (0, i))],
      out_specs=pl.BlockSpec((CB // 2, 128), lambda i: (i, 0)),
      out_shape=jax.ShapeDtypeStruct((VPAIR, 128), jnp.float32),
  )(table_t)


NC = 2
NS = 16
NW = NC * NS
B_PER_W = B // NW
RPW = B_PER_W * L
CHUNK = 160
NCHUNK = RPW // CHUNK
NSUPER = NCHUNK // 2
NBUF = 2
KG = CHUNK // 2 // 16


def _sc_gather(
    idx_hbm, pairs_hbm, out_hbm, idx_v, glist0_v, glist1_v, sel_v, raw_v,
    stage_v, gsem, osem
):
  glists = (glist0_v, glist1_v)
  wid = lax.axis_index("s") * NC + lax.axis_index("c")
  b0 = wid * B_PER_W
  pltpu.sync_copy(idx_hbm.at[pl.ds(b0, B_PER_W)], idx_v)

  iota = lax.iota(jnp.int32, 16)

  def build_lists(g, buf):
    def step(t, _):
      p = g * CHUNK + t * 16 + iota
      r = lax.div(p, jnp.int32(L))
      c = p - r * L
      v = plsc.load_gather(idx_v, [r, c])
      bufv = jnp.int32(buf) + 0 * iota
      pr = lax.shift_right_logical(v, 13) * (CB // 2) + lax.bitwise_and(
          v, CB // 2 - 1
      )
      plsc.store_scatter(glists[buf], [t * 16 + iota], pr)
      plsc.store_scatter(
          sel_v,
          [bufv, t * 16 + iota],
          lax.shift_left(
              lax.bitwise_and(lax.shift_right_logical(v, 12), 1), 6
          ),
      )
      return 0

    lax.fori_loop(0, CHUNK // 16, step, 0)

  def gather_start(g, buf):
    build_lists(g, buf)
    pltpu.async_copy(pairs_hbm.at[glists[buf]], raw_v.at[buf], gsem)

  def gather_wait(buf):
    pltpu.make_async_copy(
        pairs_hbm.at[glists[buf]], raw_v.at[buf], gsem
    ).wait()

  def select(buf):
    def grp_step(t, _):
      offv = sel_v[buf, pl.ds(16 * t, 16)]
      for i in range(16):
        r = 16 * t + i
        off = offv[i]
        for q in range(DIM // 16):
          stage_v[buf, 8 * t + i // 2, pl.ds((i % 2) * DIM + 16 * q, 16)] = (
              raw_v[buf, r, pl.ds(off + 16 * q, 16)]
          )
      return 0

    lax.fori_loop(0, CHUNK // 16, grp_step, 0)

  def out_start(g, buf):
    pltpu.async_copy(
        stage_v.at[buf],
        out_hbm.at[pl.ds(wid * (RPW // 2) + g * (CHUNK // 2), CHUNK // 2)],
        osem,
    )

  def out_wait(buf):
    pltpu.make_async_copy(
        stage_v.at[buf],
        out_hbm.at[pl.ds(wid * (RPW // 2), CHUNK // 2)],
        osem,
    ).wait()

  gather_start(0, 0)

  def super_step(s, _):
    g0 = 2 * s
    gather_wait(0)
    gather_start(g0 + 1, 1)

    @pl.when(s >= 1)
    def _():
      out_wait(0)

    select(0)
    out_start(g0, 0)

    gather_wait(1)

    @pl.when(s + 1 < NSUPER)
    def _():
      gather_start(g0 + 2, 0)

    @pl.when(s >= 1)
    def _():
      out_wait(1)

    select(1)
    out_start(g0 + 1, 1)
    return 0

  lax.fori_loop(0, NSUPER, super_step, 0)
  out_wait(0)
  out_wait(1)


@jax.jit
def _embedding(idx2d, table):
  pairs = _repack(table.T)
  mesh = plsc.VectorSubcoreMesh(core_axis_name="c", subcore_axis_name="s")
  f = pl.kernel(
      _sc_gather,
      out_type=jax.ShapeDtypeStruct((N // 2, 128), jnp.float32),
      mesh=mesh,
      scratch_types=[
          pltpu.VMEM((B_PER_W, L), jnp.int32),
          pltpu.VMEM((CHUNK,), jnp.int32),
          pltpu.VMEM((CHUNK,), jnp.int32),
          pltpu.VMEM((NBUF, CHUNK), jnp.int32),
          pltpu.VMEM((NBUF, CHUNK, 128), jnp.float32),
          pltpu.VMEM((NBUF, CHUNK // 2, 128), jnp.float32),
          pltpu.SemaphoreType.DMA,
          pltpu.SemaphoreType.DMA,
      ],
      compiler_params=pltpu.CompilerParams(
          use_tc_tiling_on_sc=True, needs_layout_passes=False
      ),
  )
  out_pairs = f(idx2d, pairs)
  return out_pairs.reshape(B, L, DIM)


def kernel(input, table):
  return _embedding(input.astype(jnp.int32), table)


# speedup vs baseline: 1.2000x; 1.0657x over previous
import functools

import jax
import jax.numpy as jnp
from jax import lax
from jax.experimental import pallas as pl
from jax.experimental.pallas import tpu as pltpu
from jax.experimental.pallas import tpu_sc as plsc

B = 4096
L = 50
DIM = 64
N = B * L
V = 1_000_000


CB = 16384


NBLK = pl.cdiv(V, CB)
VPAIR = NBLK * (CB // 2)


def _tc_repack(tt_ref, out_ref):
  x = tt_ref[...]
  y = jnp.swapaxes(x, 0, 1)
  out_ref[...] = jnp.concatenate([y[: CB // 2], y[CB // 2 :]], axis=1)


def _repack(table_t):
  return pl.pallas_call(
      _tc_repack,
      grid=(NBLK,),
      in_specs=[pl.BlockSpec((DIM, CB), lambda i: (0, i))],
      out_specs=pl.BlockSpec((CB // 2, 128), lambda i: (i, 0)),
      out_shape=jax.ShapeDtypeStruct((VPAIR, 128), jnp.float32),
  )(table_t)


NC = 2
NS = 16
NW = NC * NS
B_PER_W = B // NW
RPW = B_PER_W * L
CHUNK = 160
NCHUNK = RPW // CHUNK
NSUPER = NCHUNK // 2
NBUF = 2
KG = CHUNK // 2 // 16


def _sc_gather(
    idx_hbm, pairs_hbm, out_hbm, idx_v, glist0_v, glist1_v, sel_v, raw_v,
    stage_v, gsem, osem
):
  glists = (glist0_v, glist1_v)
  wid = lax.axis_index("s") * NC + lax.axis_index("c")
  b0 = wid * B_PER_W
  pltpu.sync_copy(idx_hbm.at[pl.ds(b0, B_PER_W)], idx_v)

  iota = lax.iota(jnp.int32, 16)

  def build_lists(g, buf):
    def step(t, _):
      p = g * CHUNK + t * 16 + iota
      r = lax.div(p, jnp.int32(L))
      c = p - r * L
      v = plsc.load_gather(idx_v, [r, c])
      bufv = jnp.int32(buf) + 0 * iota
      pr = lax.shift_right_logical(v, 14) * (CB // 2) + lax.bitwise_and(
          v, CB // 2 - 1
      )
      plsc.store_scatter(glists[buf], [t * 16 + iota], pr)
      plsc.store_scatter(
          sel_v,
          [bufv, t * 16 + iota],
          lax.shift_left(
              lax.bitwise_and(lax.shift_right_logical(v, 13), 1), 6
          ),
      )
      return 0

    lax.fori_loop(0, CHUNK // 16, step, 0)

  def gather_start(g, buf):
    build_lists(g, buf)
    pltpu.async_copy(pairs_hbm.at[glists[buf]], raw_v.at[buf], gsem)

  def gather_wait(buf):
    pltpu.make_async_copy(
        pairs_hbm.at[glists[buf]], raw_v.at[buf], gsem
    ).wait()

  def select(buf):
    def grp_step(t, _):
      offv = sel_v[buf, pl.ds(16 * t, 16)]
      for i in range(16):
        r = 16 * t + i
        off = offv[i]
        for q in range(DIM // 16):
          stage_v[buf, 8 * t + i // 2, pl.ds((i % 2) * DIM + 16 * q, 16)] = (
              raw_v[buf, r, pl.ds(off + 16 * q, 16)]
          )
      return 0

    lax.fori_loop(0, CHUNK // 16, grp_step, 0)

  def out_start(g, buf):
    pltpu.async_copy(
        stage_v.at[buf],
        out_hbm.at[pl.ds(wid * (RPW // 2) + g * (CHUNK // 2), CHUNK // 2)],
        osem,
    )

  def out_wait(buf):
    pltpu.make_async_copy(
        stage_v.at[buf],
        out_hbm.at[pl.ds(wid * (RPW // 2), CHUNK // 2)],
        osem,
    ).wait()

  gather_start(0, 0)

  def super_step(s, _):
    g0 = 2 * s
    gather_wait(0)
    gather_start(g0 + 1, 1)

    @pl.when(s >= 1)
    def _():
      out_wait(0)

    select(0)
    out_start(g0, 0)

    gather_wait(1)

    @pl.when(s + 1 < NSUPER)
    def _():
      gather_start(g0 + 2, 0)

    @pl.when(s >= 1)
    def _():
      out_wait(1)

    select(1)
    out_start(g0 + 1, 1)
    return 0

  lax.fori_loop(0, NSUPER, super_step, 0)
  out_wait(0)
  out_wait(1)


@jax.jit
def _embedding(idx2d, table):
  pairs = _repack(table.T)
  mesh = plsc.VectorSubcoreMesh(core_axis_name="c", subcore_axis_name="s")
  f = pl.kernel(
      _sc_gather,
      out_type=jax.ShapeDtypeStruct((N // 2, 128), jnp.float32),
      mesh=mesh,
      scratch_types=[
          pltpu.VMEM((B_PER_W, L), jnp.int32),
          pltpu.VMEM((CHUNK,), jnp.int32),
          pltpu.VMEM((CHUNK,), jnp.int32),
          pltpu.VMEM((NBUF, CHUNK), jnp.int32),
          pltpu.VMEM((NBUF, CHUNK, 128), jnp.float32),
          pltpu.VMEM((NBUF, CHUNK // 2, 128), jnp.float32),
          pltpu.SemaphoreType.DMA,
          pltpu.SemaphoreType.DMA,
      ],
      compiler_params=pltpu.CompilerParams(
          use_tc_tiling_on_sc=True, needs_layout_passes=False
      ),
  )
  out_pairs = f(idx2d, pairs)
  return out_pairs.reshape(B, L, DIM)


def kernel(input, table):
  return _embedding(input.astype(jnp.int32), table)


# speedup vs baseline: 1.2422x; 1.0351x over previous
import functools

import jax
import jax.numpy as jnp
from jax import lax
from jax.experimental import pallas as pl
from jax.experimental.pallas import tpu as pltpu
from jax.experimental.pallas import tpu_sc as plsc

B = 4096
L = 50
DIM = 64
N = B * L
V = 1_000_000


CB = 32768


NBLK = pl.cdiv(V, CB)
VPAIR = NBLK * (CB // 2)


def _tc_repack(tt_ref, out_ref):
  x = tt_ref[...]
  y = jnp.swapaxes(x, 0, 1)
  out_ref[...] = jnp.concatenate([y[: CB // 2], y[CB // 2 :]], axis=1)


def _repack(table_t):
  return pl.pallas_call(
      _tc_repack,
      grid=(NBLK,),
      in_specs=[pl.BlockSpec((DIM, CB), lambda i: (0, i))],
      out_specs=pl.BlockSpec((CB // 2, 128), lambda i: (i, 0)),
      out_shape=jax.ShapeDtypeStruct((VPAIR, 128), jnp.float32),
  )(table_t)


NC = 2
NS = 16
NW = NC * NS
B_PER_W = B // NW
RPW = B_PER_W * L
CHUNK = 160
NCHUNK = RPW // CHUNK
NSUPER = NCHUNK // 2
NBUF = 2
KG = CHUNK // 2 // 16


def _sc_gather(
    idx_hbm, pairs_hbm, out_hbm, idx_v, glist0_v, glist1_v, sel_v, raw_v,
    stage_v, gsem, osem
):
  glists = (glist0_v, glist1_v)
  wid = lax.axis_index("s") * NC + lax.axis_index("c")
  b0 = wid * B_PER_W
  pltpu.sync_copy(idx_hbm.at[pl.ds(b0, B_PER_W)], idx_v)

  iota = lax.iota(jnp.int32, 16)

  def build_lists(g, buf):
    def step(t, _):
      p = g * CHUNK + t * 16 + iota
      r = lax.div(p, jnp.int32(L))
      c = p - r * L
      v = plsc.load_gather(idx_v, [r, c])
      bufv = jnp.int32(buf) + 0 * iota
      pr = lax.shift_right_logical(v, 15) * (CB // 2) + lax.bitwise_and(
          v, CB // 2 - 1
      )
      plsc.store_scatter(glists[buf], [t * 16 + iota], pr)
      plsc.store_scatter(
          sel_v,
          [bufv, t * 16 + iota],
          lax.shift_left(
              lax.bitwise_and(lax.shift_right_logical(v, 14), 1), 6
          ),
      )
      return 0

    lax.fori_loop(0, CHUNK // 16, step, 0)

  def gather_start(g, buf):
    build_lists(g, buf)
    pltpu.async_copy(pairs_hbm.at[glists[buf]], raw_v.at[buf], gsem)

  def gather_wait(buf):
    pltpu.make_async_copy(
        pairs_hbm.at[glists[buf]], raw_v.at[buf], gsem
    ).wait()

  def select(buf):
    def grp_step(t, _):
      offv = sel_v[buf, pl.ds(16 * t, 16)]
      for i in range(16):
        r = 16 * t + i
        off = offv[i]
        for q in range(DIM // 16):
          stage_v[buf, 8 * t + i // 2, pl.ds((i % 2) * DIM + 16 * q, 16)] = (
              raw_v[buf, r, pl.ds(off + 16 * q, 16)]
          )
      return 0

    lax.fori_loop(0, CHUNK // 16, grp_step, 0)

  def out_start(g, buf):
    pltpu.async_copy(
        stage_v.at[buf],
        out_hbm.at[pl.ds(wid * (RPW // 2) + g * (CHUNK // 2), CHUNK // 2)],
        osem,
    )

  def out_wait(buf):
    pltpu.make_async_copy(
        stage_v.at[buf],
        out_hbm.at[pl.ds(wid * (RPW // 2), CHUNK // 2)],
        osem,
    ).wait()

  gather_start(0, 0)

  def super_step(s, _):
    g0 = 2 * s
    gather_wait(0)
    gather_start(g0 + 1, 1)

    @pl.when(s >= 1)
    def _():
      out_wait(0)

    select(0)
    out_start(g0, 0)

    gather_wait(1)

    @pl.when(s + 1 < NSUPER)
    def _():
      gather_start(g0 + 2, 0)

    @pl.when(s >= 1)
    def _():
      out_wait(1)

    select(1)
    out_start(g0 + 1, 1)
    return 0

  lax.fori_loop(0, NSUPER, super_step, 0)
  out_wait(0)
  out_wait(1)


@jax.jit
def _embedding(idx2d, table):
  pairs = _repack(table.T)
  mesh = plsc.VectorSubcoreMesh(core_axis_name="c", subcore_axis_name="s")
  f = pl.kernel(
      _sc_gather,
      out_type=jax.ShapeDtypeStruct((N // 2, 128), jnp.float32),
      mesh=mesh,
      scratch_types=[
          pltpu.VMEM((B_PER_W, L), jnp.int32),
          pltpu.VMEM((CHUNK,), jnp.int32),
          pltpu.VMEM((CHUNK,), jnp.int32),
          pltpu.VMEM((NBUF, CHUNK), jnp.int32),
          pltpu.VMEM((NBUF, CHUNK, 128), jnp.float32),
          pltpu.VMEM((NBUF, CHUNK // 2, 128), jnp.float32),
          pltpu.SemaphoreType.DMA,
          pltpu.SemaphoreType.DMA,
      ],
      compiler_params=pltpu.CompilerParams(
          use_tc_tiling_on_sc=True, needs_layout_passes=False
      ),
  )
  out_pairs = f(idx2d, pairs)
  return out_pairs.reshape(B, L, DIM)


def kernel(input, table):
  return _embedding(input.astype(jnp.int32), table)
